# baseline (speedup 1.0000x reference)
import functools
import math

import jax
import jax.numpy as jnp
from jax import lax
from jax.experimental import pallas as pl
from jax.experimental.pallas import tpu as pltpu
from jax.experimental.pallas import tpu_sc as plsc

DIM = 64
SCALE = math.sqrt(DIM)
NBUF = 2
C = 256

_info = plsc.get_sparse_core_info()
_NC = _info.num_cores
_NS = _info.num_subcores
_NW = _NC * _NS


@functools.lru_cache(maxsize=None)
def _make_emb(B1: int, B2: int, V2: int):
    nblk = B1 // C
    total = B2 * nblk
    assert total % _NW == 0
    per_w = total // _NW
    assert per_w % NBUF == 0 and per_w > NBUF
    mesh = plsc.VectorSubcoreMesh(core_axis_name="c", subcore_axis_name="s")

    @functools.partial(
        pl.kernel,
        out_type=jax.ShapeDtypeStruct((B2, DIM, B1), jnp.float32),
        mesh=mesh,
        scratch_types=[
            [pltpu.VMEM((C,), jnp.int32) for _ in range(NBUF)],
            [pltpu.VMEM((C,), jnp.int32) for _ in range(NBUF)],
            [pltpu.VMEM((C,), jnp.int32) for _ in range(NBUF)],
            [pltpu.VMEM((C, 128), jnp.float32) for _ in range(NBUF)],
            [pltpu.VMEM((DIM, C + 1), jnp.float32) for _ in range(NBUF)],
            [pltpu.SemaphoreType.DMA for _ in range(NBUF)],
            [pltpu.SemaphoreType.DMA for _ in range(NBUF)],
            [pltpu.SemaphoreType.DMA for _ in range(NBUF)],
        ],
        compiler_params=pltpu.CompilerParams(needs_layout_passes=False),
    )
    def emb(x_hbm, lut_hbm, out_hbm, idx_v, pair_v, par_v, rows_v, outt_v,
            isem, gsem, osem):
        wid = lax.axis_index("s") * _NC + lax.axis_index("c")
        cid0 = wid * per_w
        iota = lax.iota(jnp.int32, 16)

        def coords(j):
            cid = cid0 + j
            b2 = cid // nblk
            return b2, (cid - b2 * nblk) * C

        def idx_dma(j, b):
            b2, b1 = coords(j)
            return pltpu.make_async_copy(
                x_hbm.at[b2, pl.ds(b1, C)], idx_v[b], isem[b])

        def gather(b):
            return pltpu.make_async_copy(
                lut_hbm.at[pair_v[b]], rows_v[b], gsem[b])

        def out_dma(j, b):
            b2, b1 = coords(j)
            return pltpu.make_async_copy(
                outt_v[b].at[:, pl.ds(0, C)],
                out_hbm.at[b2, pl.ds(0, DIM), pl.ds(b1, C)], osem[b])

        def make_pairs(b):
            @plsc.parallel_loop(0, C // 16, unroll=4)
            def _(i):
                sl = pl.ds(i * 16, 16)
                iv = idx_v[b][sl]
                pair_v[b][sl] = lax.shift_right_logical(iv, 1)
                par_v[b][sl] = (iv & 1) << 6

        idx_dma(jnp.int32(0), 0).start()
        idx_dma(jnp.int32(1), 1).start()
        idx_dma(jnp.int32(0), 0).wait()
        make_pairs(0)
        gather(0).start()

        def outer(j2, carry):
            for b in range(NBUF):
                j = j2 * NBUF + b
                nb = 1 - b
                gather(b).wait()

                @pl.when(j + 1 < per_w)
                def _():
                    idx_dma(j + 1, nb).wait()
                    make_pairs(nb)
                    gather(nb).start()

                @pl.when(j + 2 < per_w)
                def _():
                    idx_dma(j + 2, b).start()

                @pl.when(j >= NBUF)
                def _():
                    out_dma(j - NBUF, b).wait()

                @plsc.parallel_loop(0, C // 16, unroll=1)
                def _(i):
                    t0 = i * 16
                    pv = par_v[b][pl.ds(t0, 16)]
                    for k in range(16):
                        cs = pv[k]
                        tv = lax.broadcast(t0 + k, (16,))
                        for q in range(DIM // 16):
                            v = rows_v[b][t0 + k, pl.ds(cs + 16 * q, 16)]
                            plsc.store_scatter(
                                outt_v[b], [iota + 16 * q, tv], v * SCALE)

                out_dma(j, b).start()
            return carry

        lax.fori_loop(0, per_w // NBUF, outer, 0)
        for b in range(NBUF):
            out_dma(jnp.int32(per_w - NBUF + b), b).wait()

    return emb


def kernel(x, lut):
    B1, B2 = x.shape
    V = lut.shape[0]
    xT = x.T
    lut2 = lut.reshape(V // 2, 2 * DIM)
    out = _make_emb(B1, B2, V // 2)(xT, lut2)
    return jnp.transpose(out, (2, 0, 1))


# speedup vs baseline: 1.0025x; 1.0025x over previous
import functools
import math

import jax
import jax.numpy as jnp
from jax import lax
from jax.experimental import pallas as pl
from jax.experimental.pallas import tpu as pltpu
from jax.experimental.pallas import tpu_sc as plsc

DIM = 64
SCALE = math.sqrt(DIM)
NBUF = 2
C = 256

_info = plsc.get_sparse_core_info()
_NC = _info.num_cores
_NS = _info.num_subcores
_NW = _NC * _NS


@functools.lru_cache(maxsize=None)
def _make_emb(B1: int, B2: int, V2: int):
    nblk = B1 // C
    total = B2 * nblk
    assert total % _NW == 0
    per_w = total // _NW
    assert per_w % NBUF == 0 and per_w > NBUF
    mesh = plsc.VectorSubcoreMesh(core_axis_name="c", subcore_axis_name="s")

    @functools.partial(
        pl.kernel,
        out_type=jax.ShapeDtypeStruct((B2, DIM, B1), jnp.float32),
        mesh=mesh,
        scratch_types=[
            [pltpu.VMEM((C,), jnp.int32) for _ in range(NBUF)],
            [pltpu.VMEM((C, 128), jnp.float32) for _ in range(NBUF)],
            [pltpu.VMEM((DIM, C + 1), jnp.float32) for _ in range(NBUF)],
            [pltpu.SemaphoreType.DMA for _ in range(NBUF)],
            [pltpu.SemaphoreType.DMA for _ in range(NBUF)],
            [pltpu.SemaphoreType.DMA for _ in range(NBUF)],
        ],
        compiler_params=pltpu.CompilerParams(needs_layout_passes=False),
    )
    def emb(x_hbm, lut_hbm, out_hbm, idx_v, rows_v, outt_v,
            isem, gsem, osem):
        wid = lax.axis_index("s") * _NC + lax.axis_index("c")
        cid0 = wid * per_w
        iota = lax.iota(jnp.int32, 16)

        def coords(j):
            cid = cid0 + j
            b2 = cid // nblk
            return b2, (cid - b2 * nblk) * C

        def idx_dma(j, b):
            b2, b1 = coords(j)
            return pltpu.make_async_copy(
                x_hbm.at[b2, pl.ds(b1, C)], idx_v[b], isem[b])

        def gather(b):
            return pltpu.make_async_copy(
                lut_hbm.at[idx_v[b]], rows_v[b], gsem[b])

        def out_dma(j, b):
            b2, b1 = coords(j)
            return pltpu.make_async_copy(
                outt_v[b].at[:, pl.ds(0, C)],
                out_hbm.at[b2, pl.ds(0, DIM), pl.ds(b1, C)], osem[b])

        idx_dma(jnp.int32(0), 0).start()
        idx_dma(jnp.int32(1), 1).start()
        idx_dma(jnp.int32(0), 0).wait()
        gather(0).start()

        def outer(j2, carry):
            for b in range(NBUF):
                j = j2 * NBUF + b
                nb = 1 - b
                gather(b).wait()

                @pl.when(j + 1 < per_w)
                def _():
                    idx_dma(j + 1, nb).wait()
                    gather(nb).start()

                @pl.when(j + 2 < per_w)
                def _():
                    idx_dma(j + 2, b).start()

                @pl.when(j >= NBUF)
                def _():
                    out_dma(j - NBUF, b).wait()

                @plsc.parallel_loop(0, C, unroll=4)
                def _(t):
                    tv = lax.broadcast(t, (16,))
                    for q in range(DIM // 16):
                        v = rows_v[b][t, pl.ds(16 * q, 16)]
                        plsc.store_scatter(
                            outt_v[b], [iota + 16 * q, tv], v * SCALE)

                out_dma(j, b).start()
            return carry

        lax.fori_loop(0, per_w // NBUF, outer, 0)
        for b in range(NBUF):
            out_dma(jnp.int32(per_w - NBUF + b), b).wait()

    return emb


def kernel(x, lut):
    B1, B2 = x.shape
    V = lut.shape[0]
    xT = x.T
    lut2 = jnp.concatenate([lut, lut], axis=1)
    out = _make_emb(B1, B2, V)(xT, lut2)
    return jnp.transpose(out, (2, 0, 1))


# speedup vs baseline: 1.1367x; 1.1339x over previous
import functools
import math

import jax
import jax.numpy as jnp
from jax import lax
from jax.experimental import pallas as pl
from jax.experimental.pallas import tpu as pltpu
from jax.experimental.pallas import tpu_sc as plsc

DIM = 64
SCALE = math.sqrt(DIM)
NBUF = 2

_info = plsc.get_sparse_core_info()
_NC = _info.num_cores
_NS = _info.num_subcores
_NW = _NC * _NS


@functools.lru_cache(maxsize=None)
def _make_emb(B: int, C: int):
    assert B % _NW == 0
    b_per_w = B // _NW
    assert b_per_w % C == 0 and C % 8 == 0
    nch = b_per_w // C
    assert nch % NBUF == 0 and nch > NBUF
    mesh = plsc.VectorSubcoreMesh(core_axis_name="c", subcore_axis_name="s")

    @functools.partial(
        pl.kernel,
        out_type=jax.ShapeDtypeStruct((B, DIM), jnp.float32),
        mesh=mesh,
        scratch_types=[
            pltpu.VMEM((nch, C), jnp.int32),
            [pltpu.VMEM((C, DIM), jnp.float32) for _ in range(NBUF)],
            [pltpu.VMEM((C, DIM), jnp.float32) for _ in range(NBUF)],
            [pltpu.SemaphoreType.DMA for _ in range(NBUF)],
            [pltpu.SemaphoreType.DMA for _ in range(NBUF)],
        ],
        compiler_params=pltpu.CompilerParams(use_tc_tiling_on_sc=False),
    )
    def emb(x_hbm, lut_hbm, out_hbm, idx_v, rows_in, rows_out, gsem, osem):
        wid = lax.axis_index("s") * _NC + lax.axis_index("c")
        base = wid * b_per_w
        pltpu.sync_copy(x_hbm.at[pl.ds(wid * nch, nch)], idx_v)

        def gather(g, b):
            return pltpu.make_async_copy(
                lut_hbm.at[idx_v.at[g]], rows_in[b], gsem[b])

        def scatter(g, b):
            return pltpu.make_async_copy(
                rows_out[b], out_hbm.at[pl.ds(base + g * C, C)], osem[b])

        for b in range(NBUF):
            gather(jnp.int32(b), b).start()

        def outer(i, carry):
            go = i * NBUF
            for b in range(NBUF):
                g = go + b
                gather(g, b).wait()

                @pl.when(g >= NBUF)
                def _():
                    scatter(g - NBUF, b).wait()

                @plsc.parallel_loop(0, C, unroll=8)
                def _(r):
                    for q in range(DIM // 16):
                        sl = pl.ds(q * 16, 16)
                        rows_out[b][r, sl] = rows_in[b][r, sl] * SCALE

                @pl.when(g + NBUF < nch)
                def _():
                    gather(g + NBUF, b).start()

                scatter(g, b).start()
            return carry

        lax.fori_loop(0, nch // NBUF, outer, 0)
        for b in range(NBUF):
            scatter(jnp.int32(nch - NBUF + b), b).wait()

    return emb


def kernel(x, lut):
    B = x.size
    C = 400
    out = _make_emb(B, C)(x.reshape(B // C, C), lut)
    return out.reshape(*x.shape, DIM)
